# baseline (speedup 1.0000x reference)
import functools

import jax
import jax.numpy as jnp
from jax import lax
from jax.experimental import pallas as pl
from jax.experimental.pallas import tpu as pltpu
from jax.experimental.pallas import tpu_sc as plsc

_B = 16384
_J = 50
_D = 32
_N = _B * _J


def _make_gather4():
    info = plsc.get_sparse_core_info()
    nw = info.num_cores * info.num_subcores
    npw = _N // nw
    chunk = 400
    n_chunks = npw // chunk

    mesh = plsc.VectorSubcoreMesh(core_axis_name="c", subcore_axis_name="s")

    @functools.partial(
        pl.kernel,
        mesh=mesh,
        out_type=jax.ShapeDtypeStruct((_N * _D,), jnp.float32),
        scratch_types=[
            pltpu.VMEM((npw,), jnp.int32),
            pltpu.VMEM((npw,), jnp.int32),
            pltpu.VMEM((chunk, 128), jnp.float32),
            pltpu.VMEM((chunk * _D,), jnp.float32),
            pltpu.SemaphoreType.DMA,
        ],
        compiler_params=pltpu.CompilerParams(needs_layout_passes=False),
    )
    def gather_kernel(idx_hbm, w4_hbm, out_hbm, idx_v, idx4_v, rows_v, out_v, sem):
        wid = lax.axis_index("s") * info.num_cores + lax.axis_index("c")
        base = wid * npw
        pltpu.sync_copy(idx_hbm.at[pl.ds(base, npw)], idx_v)

        def div_body(k, carry):
            v = idx_v[pl.ds(k * 16, 16)]
            idx4_v[pl.ds(k * 16, 16)] = lax.shift_right_logical(v, 2)
            return carry

        lax.fori_loop(0, npw // 16, div_body, 0)

        lanes = lax.iota(jnp.int32, 16)

        def chunk_body(g, carry):
            off = pl.multiple_of(g * chunk, chunk)
            pltpu.async_copy(
                w4_hbm.at[idx4_v.at[pl.ds(off, chunk)]], rows_v, sem
            ).wait()

            def ext_body(t, carry2):
                row0 = t * 16
                iq = idx_v[pl.ds(off + row0, 16)]
                col0 = (iq & 3) * _D
                rowv = lanes + row0
                dst0 = rowv * _D
                for c in range(_D):
                    v = plsc.load_gather(rows_v, [rowv, col0 + c])
                    plsc.store_scatter(out_v, [dst0 + c], v)
                return carry2

            lax.fori_loop(0, chunk // 16, ext_body, 0)
            pltpu.sync_copy(
                out_v, out_hbm.at[pl.ds((base + off) * _D, chunk * _D)]
            )
            return carry

        lax.fori_loop(0, n_chunks, chunk_body, 0)

    return gather_kernel


_gather4 = _make_gather4()


def kernel(indices, weight):
    flat_idx = indices.reshape(-1)
    w4 = weight.reshape(250000, 128)
    out1d = _gather4(flat_idx, w4)
    return out1d.reshape(_B, _J, _D)


# speedup vs baseline: 1.1852x; 1.1852x over previous
import functools

import jax
import jax.numpy as jnp
from jax import lax
from jax.experimental import pallas as pl
from jax.experimental.pallas import tpu as pltpu
from jax.experimental.pallas import tpu_sc as plsc

_B = 16384
_J = 50
_D = 32
_N = _B * _J


def _make_gather4():
    info = plsc.get_sparse_core_info()
    nw = info.num_cores * info.num_subcores
    npw = _N // nw
    chunk = 400
    n_chunks = npw // chunk

    mesh = plsc.VectorSubcoreMesh(core_axis_name="c", subcore_axis_name="s")

    @functools.partial(
        pl.kernel,
        mesh=mesh,
        out_type=jax.ShapeDtypeStruct((_N * _D,), jnp.float32),
        scratch_types=[
            pltpu.VMEM((npw,), jnp.int32),
            pltpu.VMEM((npw,), jnp.int32),
            pltpu.VMEM((chunk, 128), jnp.float32),
            pltpu.VMEM((chunk * _D,), jnp.float32),
            pltpu.SemaphoreType.DMA,
        ],
        compiler_params=pltpu.CompilerParams(needs_layout_passes=False),
    )
    def gather_kernel(idx_hbm, w4_hbm, out_hbm, idx_v, idx4_v, rows_v, out_v, sem):
        wid = lax.axis_index("s") * info.num_cores + lax.axis_index("c")
        base = wid * npw
        pltpu.sync_copy(idx_hbm.at[pl.ds(base, npw)], idx_v)

        @plsc.parallel_loop(0, npw // 16, unroll=4)
        def div_body(k):
            v = idx_v[pl.ds(k * 16, 16)]
            idx4_v[pl.ds(k * 16, 16)] = lax.shift_right_logical(v, 2)

        lanes = lax.iota(jnp.int32, 16)

        def chunk_body(g, carry):
            off = pl.multiple_of(g * chunk, chunk)
            pltpu.async_copy(
                w4_hbm.at[idx4_v.at[pl.ds(off, chunk)]], rows_v, sem
            ).wait()

            @plsc.parallel_loop(0, chunk // 16, unroll=2)
            def ext_body(t):
                row0 = t * 16
                iq = idx_v[pl.ds(off + row0, 16)]
                col0 = (iq & 3) * _D
                rowv = lanes + row0
                dst0 = rowv * _D
                for c in range(_D):
                    v = plsc.load_gather(rows_v, [rowv, col0 + c])
                    plsc.store_scatter(out_v, [dst0 + c], v)
            pltpu.sync_copy(
                out_v, out_hbm.at[pl.ds((base + off) * _D, chunk * _D)]
            )
            return carry

        lax.fori_loop(0, n_chunks, chunk_body, 0)

    return gather_kernel


_gather4 = _make_gather4()


def kernel(indices, weight):
    flat_idx = indices.reshape(-1)
    w4 = weight.reshape(250000, 128)
    out1d = _gather4(flat_idx, w4)
    return out1d.reshape(_B, _J, _D)


# speedup vs baseline: 1.9981x; 1.6858x over previous
import functools

import jax
import jax.numpy as jnp
from jax import lax
from jax.experimental import pallas as pl
from jax.experimental.pallas import tpu as pltpu
from jax.experimental.pallas import tpu_sc as plsc

_B = 16384
_J = 50
_D = 32


def _make_gather():
    info = plsc.get_sparse_core_info()
    nw = info.num_cores * info.num_subcores
    b_per_w = _B // nw
    cb = 32
    chunk = cb * _J
    n_chunks = b_per_w // cb

    mesh = plsc.VectorSubcoreMesh(core_axis_name="c", subcore_axis_name="s")

    @functools.partial(
        pl.kernel,
        mesh=mesh,
        out_type=jax.ShapeDtypeStruct((_B, _J, _D), jnp.float32),
        scratch_types=[
            pltpu.VMEM((b_per_w * _J,), jnp.int32),
            pltpu.VMEM((chunk, _D), jnp.float32),
            pltpu.SemaphoreType.DMA,
        ],
        compiler_params=pltpu.CompilerParams(use_tc_tiling_on_sc=False),
    )
    def gather_kernel(idx_hbm, table_hbm, out_hbm, idx_v, rows_v, sem):
        wid = lax.axis_index("s") * info.num_cores + lax.axis_index("c")
        base = wid * b_per_w * _J
        pltpu.sync_copy(idx_hbm.at[pl.ds(base, b_per_w * _J)], idx_v)

        def body(g, carry):
            off = pl.multiple_of(g * chunk, chunk)
            bb = wid * b_per_w + g * cb
            pltpu.async_copy(
                table_hbm.at[idx_v.at[pl.ds(off, chunk)]], rows_v, sem
            ).wait()
            for k in range(cb):
                pltpu.sync_copy(
                    rows_v.at[pl.ds(k * _J, _J)], out_hbm.at[bb + k]
                )
            return carry

        lax.fori_loop(0, n_chunks, body, 0)

    return gather_kernel


_gather = _make_gather()


def kernel(indices, weight):
    flat_idx = indices.reshape(-1)
    return _gather(flat_idx, weight)
